# baseline (speedup 1.0000x reference)
import jax
import jax.numpy as jnp
from jax.experimental import pallas as pl
from jax.experimental.pallas import tpu as pltpu

F32 = jnp.float32
I32 = jnp.int32
BF16 = jnp.bfloat16

HEADS = 4
HALF = 256
BLOCK = 64
COL = 1024
GROUP_TOK = 32
KSEL = 1024
VALUE_DIM = 128
EG = 16

_HIGH = jax.lax.Precision.HIGHEST
_SIGN = -2147483648


def _sortable(x):
    i = jax.lax.bitcast_convert_type(x, I32)
    return jnp.where(i < 0, i ^ 0x7FFFFFFF, i)


def _qscore_body(x_ref, cwt_ref, cb_ref, lw_ref, lb_ref, rk_ref, ck_ref,
                 s_ref, c_ref, q_scr, rkt_scr, ckt_scr):
    h = pl.program_id(0)
    tt = x_ref.shape[0] // 4

    @pl.when(h == 0)
    def _qnet():
        x = x_ref[...]
        c = x.shape[1]
        w0 = cwt_ref[0:1, :]
        w1 = cwt_ref[1:2, :]
        w2 = cwt_ref[2:3, :]
        z1 = jnp.concatenate([jnp.zeros((1, c), F32), x[:-1, :]], axis=0)
        z2 = jnp.concatenate([jnp.zeros((2, c), F32), x[:-2, :]], axis=0)
        conv = z2 * w0 + z1 * w1 + x * w2 + cb_ref[...]
        q_scr[...] = (jnp.dot(conv, lw_ref[...], preferred_element_type=F32)
                      + lb_ref[...])
        rkt_scr[...] = jnp.swapaxes(rk_ref[...], 0, 1)
        ckt_scr[...] = jnp.swapaxes(ck_ref[...], 0, 1)

    q = q_scr[pl.dslice(pl.multiple_of(h * tt, tt), tt), :].reshape(-1, HALF)
    dn = (((1,), (1,)), ((), ()))
    rowsc = jax.lax.dot_general(q, rkt_scr[h], dn, preferred_element_type=F32)
    colsc = jax.lax.dot_general(q, ckt_scr[h], dn, preferred_element_type=F32)
    cmax = jnp.max(colsc, axis=1, keepdims=True)
    lanes = jax.lax.broadcasted_iota(I32, colsc.shape, 1)
    carg = jnp.min(jnp.where(colsc == cmax, lanes, colsc.shape[1]), axis=1,
                   keepdims=True)
    s_ref[0] = rowsc + cmax
    c_ref[0] = carg


def _select_body(s_ref, w_ref):
    s = s_ref[...]
    ikey = _sortable(s)
    rows = ikey.shape[0]
    hkey = jnp.right_shift(ikey, 16).astype(jnp.int16)

    def body_hi(b, u):
        bit = jnp.left_shift(jnp.int32(1), 31 - b)
        cand = u | bit
        hcand = jnp.right_shift(cand ^ _SIGN, 16).astype(jnp.int16)
        cnt = jnp.sum((hkey >= hcand).astype(I32), axis=1, keepdims=True)
        return jnp.where(cnt >= KSEL, cand, u)

    def body_lo(b, u):
        bit = jnp.left_shift(jnp.int32(1), 31 - b)
        cand = u | bit
        icand = cand ^ _SIGN
        cnt = jnp.sum((ikey >= icand).astype(I32), axis=1, keepdims=True)
        return jnp.where(cnt >= KSEL, cand, u)

    u = jax.lax.fori_loop(0, 16, body_hi, jnp.zeros((rows, 1), I32))
    u = jax.lax.fori_loop(16, 32, body_lo, u)
    thr = u ^ _SIGN
    m = ikey >= thr
    ex = jnp.where(m, jnp.exp(s), 0.0)
    rowz = jnp.sum(ex, axis=1, keepdims=True)
    grp = (jax.lax.broadcasted_iota(I32, (rows, rows), 0) // 4
           == jax.lax.broadcasted_iota(I32, (rows, rows), 1) // 4)
    gz = jax.lax.dot_general(grp.astype(F32), rowz, (((1,), (0,)), ((), ())),
                             preferred_element_type=F32, precision=_HIGH)
    w_ref[...] = ex / gz


def _value_body(w2_ref, cu_ref, v_ref, o_ref, ws_ref):
    eg = pl.program_id(0)

    @pl.when(eg == 0)
    def _build():
        for u in range(GROUP_TOK):
            wv = w2_ref[:, u * BLOCK:(u + 1) * BLOCK]
            c = cu_ref[u]
            onehot = (jax.lax.broadcasted_iota(I32, (wv.shape[0], COL), 1)
                      == c).astype(BF16)
            ws_ref[u] = jax.lax.dot_general(
                wv.astype(BF16), onehot, (((0,), (0,)), ((), ())),
                preferred_element_type=F32).astype(BF16)

    wsg = ws_ref[:, pl.dslice(pl.multiple_of(eg * EG, EG), EG), :]
    vg = v_ref[...].astype(BF16)
    for j in range(EG):
        o_ref[:, j, :] = jnp.dot(wsg[:, j, :], vg[j],
                                 preferred_element_type=F32)


def kernel(x, conv_w, conv_b, lin_w, lin_b, rowkeys, colkeys, values_w):
    B, T, C_IN = x.shape
    QD = lin_w.shape[1]
    x2 = x.reshape(T, C_IN)

    s, carg = pl.pallas_call(
        _qscore_body,
        grid=(HEADS,),
        in_specs=[
            pl.BlockSpec((T, C_IN), lambda h: (0, 0)),
            pl.BlockSpec((3, C_IN), lambda h: (0, 0)),
            pl.BlockSpec((1, C_IN), lambda h: (0, 0)),
            pl.BlockSpec((C_IN, QD), lambda h: (0, 0)),
            pl.BlockSpec((1, QD), lambda h: (0, 0)),
            pl.BlockSpec((BLOCK, HEADS, HALF), lambda h: (0, 0, 0)),
            pl.BlockSpec((COL, HEADS, HALF), lambda h: (0, 0, 0)),
        ],
        out_specs=[
            pl.BlockSpec((1, T, BLOCK), lambda h: (h, 0, 0)),
            pl.BlockSpec((1, T, 1), lambda h: (h, 0, 0)),
        ],
        out_shape=[
            jax.ShapeDtypeStruct((HEADS, T, BLOCK), F32),
            jax.ShapeDtypeStruct((HEADS, T, 1), I32),
        ],
        scratch_shapes=[
            pltpu.VMEM((T, QD), F32),
            pltpu.VMEM((HEADS, BLOCK, HALF), F32),
            pltpu.VMEM((HEADS, COL, HALF), F32),
        ],
    )(x2, conv_w.T, conv_b[None, :], lin_w, lin_b[None, :], rowkeys, colkeys)

    nrows = HEADS * BLOCK
    s2 = s.reshape(nrows, T)
    w2 = pl.pallas_call(
        _select_body,
        out_shape=jax.ShapeDtypeStruct((nrows, T), F32),
    )(s2)

    cu = carg.reshape(HEADS, BLOCK, GROUP_TOK).transpose(2, 0, 1)
    cu = cu.reshape(GROUP_TOK, nrows, 1)

    v2 = values_w.reshape(BLOCK, COL, VALUE_DIM)
    o = pl.pallas_call(
        _value_body,
        grid=(BLOCK // EG,),
        in_specs=[
            pl.BlockSpec((nrows, T), lambda e: (0, 0)),
            pl.BlockSpec((GROUP_TOK, nrows, 1), lambda e: (0, 0, 0)),
            pl.BlockSpec((EG, COL, VALUE_DIM), lambda e: (e, 0, 0)),
        ],
        out_specs=pl.BlockSpec((GROUP_TOK, EG, VALUE_DIM), lambda e: (0, e, 0)),
        out_shape=jax.ShapeDtypeStruct((GROUP_TOK, BLOCK, VALUE_DIM), F32),
        scratch_shapes=[pltpu.VMEM((GROUP_TOK, BLOCK, COL), BF16)],
    )(w2, cu, v2)

    return o.reshape(B, T, VALUE_DIM)


# speedup vs baseline: 1.0892x; 1.0892x over previous
import jax
import jax.numpy as jnp
from jax.experimental import pallas as pl
from jax.experimental.pallas import tpu as pltpu

F32 = jnp.float32
I32 = jnp.int32
BF16 = jnp.bfloat16

HEADS = 4
HALF = 256
BLOCK = 64
COL = 1024
GROUP_TOK = 32
KSEL = 1024
VALUE_DIM = 128
EG = 16

_HIGH = jax.lax.Precision.HIGHEST
_SIGN = -2147483648


def _sortable(x):
    i = jax.lax.bitcast_convert_type(x, I32)
    return jnp.where(i < 0, i ^ 0x7FFFFFFF, i)


def _qscore_body(x_ref, cwt_ref, cb_ref, lw_ref, lb_ref, rk_ref, ck_ref,
                 s_ref, c_ref, qb_scr):
    h = pl.program_id(0)
    tt = x_ref.shape[0] // 4

    @pl.when(h == 0)
    def _qnet():
        x = x_ref[...]
        c = x.shape[1]
        w0 = cwt_ref[0:1, :]
        w1 = cwt_ref[1:2, :]
        w2 = cwt_ref[2:3, :]
        z1 = jnp.concatenate([jnp.zeros((1, c), F32), x[:-1, :]], axis=0)
        z2 = jnp.concatenate([jnp.zeros((2, c), F32), x[:-2, :]], axis=0)
        conv = z2 * w0 + z1 * w1 + x * w2 + cb_ref[...]
        qb_scr[...] = (jnp.dot(conv, lw_ref[...], preferred_element_type=F32)
                       + lb_ref[...])

    q = qb_scr[pl.dslice(pl.multiple_of(h * tt, tt), tt), :].reshape(-1, HALF)
    dn = (((1,), (1,)), ((), ()))
    rowsc = jax.lax.dot_general(q, rk_ref[0], dn, preferred_element_type=F32)
    colsc = jax.lax.dot_general(q, ck_ref[0], dn, preferred_element_type=F32)
    cmax = jnp.max(colsc, axis=1, keepdims=True)
    lanes = jax.lax.broadcasted_iota(I32, colsc.shape, 1)
    carg = jnp.min(jnp.where(colsc == cmax, lanes, colsc.shape[1]), axis=1,
                   keepdims=True)
    s_ref[0] = rowsc + cmax
    c_ref[0] = carg


def _select_weights(s):
    ikey = _sortable(s)
    rows = ikey.shape[0]

    def body(b, u):
        bit = jnp.left_shift(jnp.int32(1), 31 - b)
        cand = u | bit
        icand = cand ^ _SIGN
        cnt = jnp.sum((ikey >= icand).astype(I32), axis=1, keepdims=True)
        return jnp.where(cnt >= KSEL, cand, u)

    u = jax.lax.fori_loop(0, 32, body, jnp.zeros((rows, 1), I32))
    thr = u ^ _SIGN
    m = ikey >= thr
    ex = jnp.where(m, jnp.exp(s), 0.0)
    rowz = jnp.sum(ex, axis=1, keepdims=True)
    grp = (jax.lax.broadcasted_iota(I32, (rows, rows), 0) // 4
           == jax.lax.broadcasted_iota(I32, (rows, rows), 1) // 4)
    gz = jax.lax.dot_general(grp.astype(F32), rowz, (((1,), (0,)), ((), ())),
                             preferred_element_type=F32, precision=_HIGH)
    return (ex / gz).astype(BF16)


def _value_body(s2_ref, cu_ref, v_ref, o_ref, ws_ref):
    eg = pl.program_id(0)

    @pl.when(eg == 0)
    def _build():
        w2 = _select_weights(s2_ref[...])
        for u in range(GROUP_TOK):
            wv = w2[:, u * BLOCK:(u + 1) * BLOCK]
            c = cu_ref[u]
            onehot = (jax.lax.broadcasted_iota(I32, (wv.shape[0], COL), 1)
                      == c).astype(BF16)
            ws_ref[u] = jax.lax.dot_general(
                wv, onehot, (((0,), (0,)), ((), ())),
                preferred_element_type=F32).astype(BF16)

    wsg = ws_ref[:, pl.dslice(pl.multiple_of(eg * EG, EG), EG), :]
    vg = v_ref[...].astype(BF16)
    for j in range(EG):
        o_ref[:, j, :] = jnp.dot(wsg[:, j, :], vg[j],
                                 preferred_element_type=F32)


def kernel(x, conv_w, conv_b, lin_w, lin_b, rowkeys, colkeys, values_w):
    B, T, C_IN = x.shape
    QD = lin_w.shape[1]
    x2 = x.reshape(T, C_IN)

    s, carg = pl.pallas_call(
        _qscore_body,
        grid=(HEADS,),
        in_specs=[
            pl.BlockSpec((T, C_IN), lambda h: (0, 0)),
            pl.BlockSpec((3, C_IN), lambda h: (0, 0)),
            pl.BlockSpec((1, C_IN), lambda h: (0, 0)),
            pl.BlockSpec((C_IN, QD), lambda h: (0, 0)),
            pl.BlockSpec((1, QD), lambda h: (0, 0)),
            pl.BlockSpec((1, BLOCK, HALF), lambda h: (h, 0, 0)),
            pl.BlockSpec((1, COL, HALF), lambda h: (h, 0, 0)),
        ],
        out_specs=[
            pl.BlockSpec((1, T, BLOCK), lambda h: (h, 0, 0)),
            pl.BlockSpec((1, T, 1), lambda h: (h, 0, 0)),
        ],
        out_shape=[
            jax.ShapeDtypeStruct((HEADS, T, BLOCK), F32),
            jax.ShapeDtypeStruct((HEADS, T, 1), I32),
        ],
        scratch_shapes=[pltpu.VMEM((T, QD), F32)],
    )(x2, conv_w.T, conv_b[None, :], lin_w, lin_b[None, :],
      rowkeys.transpose(1, 0, 2), colkeys.transpose(1, 0, 2))

    nrows = HEADS * BLOCK
    s2 = s.reshape(nrows, T)

    cu = carg.reshape(HEADS, BLOCK, GROUP_TOK).transpose(2, 0, 1)
    cu = cu.reshape(GROUP_TOK, nrows, 1)

    v2 = values_w.reshape(BLOCK, COL, VALUE_DIM)
    o = pl.pallas_call(
        _value_body,
        grid=(BLOCK // EG,),
        in_specs=[
            pl.BlockSpec((nrows, T), lambda e: (0, 0)),
            pl.BlockSpec((GROUP_TOK, nrows, 1), lambda e: (0, 0, 0)),
            pl.BlockSpec((EG, COL, VALUE_DIM), lambda e: (e, 0, 0)),
        ],
        out_specs=pl.BlockSpec((GROUP_TOK, EG, VALUE_DIM), lambda e: (0, e, 0)),
        out_shape=jax.ShapeDtypeStruct((GROUP_TOK, BLOCK, VALUE_DIM), F32),
        scratch_shapes=[pltpu.VMEM((GROUP_TOK, BLOCK, COL), BF16)],
    )(s2, cu, v2)

    return o.reshape(B, T, VALUE_DIM)


# speedup vs baseline: 1.1353x; 1.0424x over previous
import jax
import jax.numpy as jnp
from jax.experimental import pallas as pl
from jax.experimental.pallas import tpu as pltpu

F32 = jnp.float32
I32 = jnp.int32
BF16 = jnp.bfloat16

HEADS = 4
HALF = 256
BLOCK = 64
COL = 1024
GROUP_TOK = 32
KSEL = 1024
VALUE_DIM = 128
EG = 16

_HIGH = jax.lax.Precision.HIGHEST
_SIGN = -2147483648


def _sortable(x):
    i = jax.lax.bitcast_convert_type(x, I32)
    return jnp.where(i < 0, i ^ 0x7FFFFFFF, i)


def _qscore_body(x_ref, cw_ref, cb_ref, lw_ref, lb_ref, rk_ref, ck_ref,
                 s_ref, c_ref, qb_scr, rkt_scr, ckt_scr):
    h = pl.program_id(0)
    tt = x_ref.shape[0] // 4

    @pl.when(h == 0)
    def _qnet():
        x = x_ref[...]
        c = x.shape[1]
        cwt = jnp.swapaxes(cw_ref[...], 0, 1)
        w0 = cwt[0:1, :]
        w1 = cwt[1:2, :]
        w2 = cwt[2:3, :]
        z1 = jnp.concatenate([jnp.zeros((1, c), F32), x[:-1, :]], axis=0)
        z2 = jnp.concatenate([jnp.zeros((2, c), F32), x[:-2, :]], axis=0)
        conv = z2 * w0 + z1 * w1 + x * w2 + cb_ref[...]
        qb_scr[...] = (jnp.dot(conv, lw_ref[...], preferred_element_type=F32)
                       + lb_ref[...])
        rkt_scr[...] = jnp.swapaxes(rk_ref[...], 0, 1)
        ckt_scr[...] = jnp.swapaxes(ck_ref[...], 0, 1)

    q = qb_scr[pl.dslice(pl.multiple_of(h * tt, tt), tt), :].reshape(-1, HALF)
    dn = (((1,), (1,)), ((), ()))
    rowsc = jax.lax.dot_general(q, rkt_scr[h], dn, preferred_element_type=F32)
    colsc = jax.lax.dot_general(q, ckt_scr[h], dn, preferred_element_type=F32)
    cmax = jnp.max(colsc, axis=1, keepdims=True)
    lanes = jax.lax.broadcasted_iota(I32, colsc.shape, 1)
    carg = jnp.min(jnp.where(colsc == cmax, lanes, colsc.shape[1]), axis=1,
                   keepdims=True)
    s_ref[0] = rowsc + cmax
    c_ref[0] = carg


def _select_weights(s):
    ikey = _sortable(s)
    rows = ikey.shape[0]

    def body(b, u):
        bit = jnp.left_shift(jnp.int32(1), 31 - b)
        cand = u | bit
        icand = cand ^ _SIGN
        cnt = jnp.sum((ikey >= icand).astype(I32), axis=1, keepdims=True)
        return jnp.where(cnt >= KSEL, cand, u)

    u = jax.lax.fori_loop(0, 32, body, jnp.zeros((rows, 1), I32))
    thr = u ^ _SIGN
    m = ikey >= thr
    ex = jnp.where(m, jnp.exp(s), 0.0)
    rowz = jnp.sum(ex, axis=1, keepdims=True)
    grp = (jax.lax.broadcasted_iota(I32, (rows, rows), 0) // 4
           == jax.lax.broadcasted_iota(I32, (rows, rows), 1) // 4)
    gz = jax.lax.dot_general(grp.astype(F32), rowz, (((1,), (0,)), ((), ())),
                             preferred_element_type=F32, precision=_HIGH)
    return (ex / gz).astype(BF16)


def _value_body(s2_ref, cu_ref, v_ref, o_ref, ws_ref):
    eg = pl.program_id(0)

    @pl.when(eg == 0)
    def _build():
        w2 = _select_weights(s2_ref[...])
        for u in range(GROUP_TOK):
            wv = w2[:, u * BLOCK:(u + 1) * BLOCK]
            c = cu_ref[u]
            onehot = (jax.lax.broadcasted_iota(I32, (wv.shape[0], COL), 1)
                      == c).astype(BF16)
            ws_ref[u] = jax.lax.dot_general(
                wv, onehot, (((0,), (0,)), ((), ())),
                preferred_element_type=F32).astype(BF16)

    wsg = ws_ref[:, pl.dslice(pl.multiple_of(eg * EG, EG), EG), :]
    vg = v_ref[...].astype(BF16)
    for j in range(EG):
        o_ref[:, j, :] = jnp.dot(wsg[:, j, :], vg[j],
                                 preferred_element_type=F32)


def kernel(x, conv_w, conv_b, lin_w, lin_b, rowkeys, colkeys, values_w):
    B, T, C_IN = x.shape
    QD = lin_w.shape[1]
    x2 = x.reshape(T, C_IN)

    s, carg = pl.pallas_call(
        _qscore_body,
        grid=(HEADS,),
        in_specs=[
            pl.BlockSpec((T, C_IN), lambda h: (0, 0)),
            pl.BlockSpec((C_IN, 3), lambda h: (0, 0)),
            pl.BlockSpec((1, C_IN), lambda h: (0, 0)),
            pl.BlockSpec((C_IN, QD), lambda h: (0, 0)),
            pl.BlockSpec((1, QD), lambda h: (0, 0)),
            pl.BlockSpec((BLOCK, HEADS, HALF), lambda h: (0, 0, 0)),
            pl.BlockSpec((COL, HEADS, HALF), lambda h: (0, 0, 0)),
        ],
        out_specs=[
            pl.BlockSpec((1, T, BLOCK), lambda h: (h, 0, 0)),
            pl.BlockSpec((1, T, 1), lambda h: (h, 0, 0)),
        ],
        out_shape=[
            jax.ShapeDtypeStruct((HEADS, T, BLOCK), F32),
            jax.ShapeDtypeStruct((HEADS, T, 1), I32),
        ],
        scratch_shapes=[
            pltpu.VMEM((T, QD), F32),
            pltpu.VMEM((HEADS, BLOCK, HALF), F32),
            pltpu.VMEM((HEADS, COL, HALF), F32),
        ],
    )(x2, conv_w, conv_b[None, :], lin_w, lin_b[None, :], rowkeys, colkeys)

    nrows = HEADS * BLOCK
    s2 = s.reshape(nrows, T)

    cu = carg.reshape(HEADS, BLOCK, GROUP_TOK).transpose(2, 0, 1)
    cu = cu.reshape(GROUP_TOK, nrows, 1)

    v2 = values_w.reshape(BLOCK, COL, VALUE_DIM)
    o = pl.pallas_call(
        _value_body,
        grid=(BLOCK // EG,),
        in_specs=[
            pl.BlockSpec((nrows, T), lambda e: (0, 0)),
            pl.BlockSpec((GROUP_TOK, nrows, 1), lambda e: (0, 0, 0)),
            pl.BlockSpec((EG, COL, VALUE_DIM), lambda e: (e, 0, 0)),
        ],
        out_specs=pl.BlockSpec((GROUP_TOK, EG, VALUE_DIM), lambda e: (0, e, 0)),
        out_shape=jax.ShapeDtypeStruct((GROUP_TOK, BLOCK, VALUE_DIM), F32),
        scratch_shapes=[pltpu.VMEM((GROUP_TOK, BLOCK, COL), BF16)],
    )(s2, cu, v2)

    return o.reshape(B, T, VALUE_DIM)
